# baseline (speedup 1.0000x reference)
import functools

import jax
import jax.numpy as jnp
from jax import lax
from jax.experimental import pallas as pl
from jax.experimental.pallas import tpu as pltpu
from jax.experimental.pallas import tpu_sc as plsc

E = 8
TOPK = 2
D = 1024
F = 1024
BT = 256
TB1 = 512
NC, NS, L = 2, 16, 16
NW = NC * NS
_SC = {"t", "x", "c"}


def _gate_body(x_ref, wg_ref, xn_ref, top2_ref, gates_ref, rank_ref,
               cnt_ref, load_ref, fsum_ref, psum_ref, *, nt, t_total):
    tb = pl.program_id(0)

    @pl.when(tb == 0)
    def _init():
        fsum_ref[...] = jnp.zeros_like(fsum_ref)
        psum_ref[...] = jnp.zeros_like(psum_ref)

    x = x_ref[...]
    logits = jnp.dot(x, wg_ref[...], preferred_element_type=jnp.float32)
    m = jnp.max(logits, axis=1, keepdims=True)
    ex = jnp.exp(logits - m)
    probs = ex / jnp.sum(ex, axis=1, keepdims=True)
    iota8 = lax.broadcasted_iota(jnp.int32, probs.shape, 1)
    v1 = jnp.max(probs, axis=1, keepdims=True)
    i1 = jnp.min(jnp.where(probs >= v1, iota8, E), axis=1, keepdims=True)
    p2 = jnp.where(iota8 == i1, -1.0, probs)
    v2 = jnp.max(p2, axis=1, keepdims=True)
    i2 = jnp.min(jnp.where(p2 >= v2, iota8, E), axis=1, keepdims=True)

    e_lo = jnp.minimum(i1, i2)
    e_hi = jnp.maximum(i1, i2)
    g_lo = jnp.where(i1 < i2, v1, v2)
    g_hi = jnp.where(i1 < i2, v2, v1)

    oh_lo = (iota8 == e_lo).astype(jnp.float32)
    oh_hi = (iota8 == e_hi).astype(jnp.float32)
    comb = oh_lo + oh_hi
    ri = lax.broadcasted_iota(jnp.int32, (TB1, TB1), 0)
    ci = lax.broadcasted_iota(jnp.int32, (TB1, TB1), 1)
    ltri = (ci < ri).astype(jnp.float32)
    excl = jnp.dot(ltri, comb, preferred_element_type=jnp.float32)
    base = fsum_ref[...] + excl
    rank_lo = jnp.sum(oh_lo * base, axis=1, keepdims=True)
    rank_hi = jnp.sum(oh_hi * base, axis=1, keepdims=True)

    fsum_ref[...] += jnp.sum(comb, axis=0, keepdims=True)
    psum_ref[...] += jnp.sum(probs, axis=0, keepdims=True)

    xn_ref[...] = x * lax.rsqrt(jnp.mean(x * x, axis=1, keepdims=True) + 1e-8)
    top2_ref[...] = jnp.concatenate([e_lo, e_hi], axis=1)
    gates_ref[...] = jnp.concatenate([g_lo, g_hi], axis=1)
    rank_ref[...] = jnp.concatenate([rank_lo, rank_hi], axis=1).astype(jnp.int32)

    @pl.when(tb == nt - 1)
    def _fin():
        cnt_ref[...] = fsum_ref[...]
        load_ref[...] = (E / (t_total * t_total)) * jnp.sum(
            fsum_ref[...] * psum_ref[...], axis=1, keepdims=True)


def _finalize_body(top2_ref, rank_ref, cnt_ref,
                   pos_ref, be_ref, xblk_ref, act_ref, *, nblk, t_total):
    cnt = cnt_ref[...]
    nb = jnp.floor((cnt + (BT - 1)) * (1.0 / BT))
    bstarts = []
    acc = jnp.zeros((1, 1), jnp.float32)
    for e in range(E):
        bstarts.append(acc)
        acc = acc + nb[:, e:e + 1]
    total_b = acc

    t2 = top2_ref[...]
    rank = rank_ref[...]
    poss = []
    for ci in range(TOPK):
        ec = t2[:, ci:ci + 1]
        sel = jnp.zeros((t_total, 1), jnp.float32)
        for e in range(E):
            sel += jnp.where(ec == e, bstarts[e] * BT, 0.0)
        poss.append(rank[:, ci:ci + 1] + sel.astype(jnp.int32))
    pos_ref[...] = jnp.concatenate(poss, axis=1)

    barr = lax.broadcasted_iota(jnp.int32, (1, nblk), 1).astype(jnp.float32)
    act_ref[...] = (barr < total_b).astype(jnp.int32)
    bcl = jnp.minimum(barr, total_b - 1.0)
    xblk_ref[...] = bcl.astype(jnp.int32)
    be = jnp.zeros((1, nblk), jnp.float32)
    for e in range(E):
        be += (bstarts[e] <= bcl).astype(jnp.float32)
    be_ref[...] = (be - 1.0).astype(jnp.int32)


def _make_dispatch(t_total, p_total, ch):
    tw = t_total // NW
    nch = tw // ch
    mesh = plsc.VectorSubcoreMesh(core_axis_name="c", subcore_axis_name="s")

    @functools.partial(
        pl.kernel, mesh=mesh,
        out_type=[jax.ShapeDtypeStruct((p_total, D), jnp.float32),
                  jax.ShapeDtypeStruct((p_total,), jnp.float32)],
        scratch_types=[pltpu.VMEM((tw,), jnp.int32),
                       pltpu.VMEM((tw,), jnp.int32),
                       pltpu.VMEM((tw,), jnp.float32),
                       pltpu.VMEM((tw,), jnp.float32),
                       pltpu.VMEM((ch, D), jnp.float32),
                       pltpu.VMEM((ch, D), jnp.float32),
                       pltpu.SemaphoreType.DMA,
                       pltpu.SemaphoreType.DMA,
                       pltpu.SemaphoreType.DMA,
                       pltpu.SemaphoreType.DMA,
                       pltpu.SemaphoreType.DMA,
                       pltpu.SemaphoreType.DMA],
    )
    def k(xn_hbm, pos0_hbm, pos1_hbm, g0_hbm, g1_hbm, xs_hbm, gs_hbm,
          p0v, p1v, g0v, g1v, rows0, rows1, sg0, sg1, sa0, sa1, sb0, sb1):
        wid = lax.axis_index("s") * NC + lax.axis_index("c")
        base = wid * tw
        pltpu.sync_copy(pos0_hbm.at[pl.ds(base, tw)], p0v)
        pltpu.sync_copy(pos1_hbm.at[pl.ds(base, tw)], p1v)
        pltpu.sync_copy(g0_hbm.at[pl.ds(base, tw)], g0v)
        pltpu.sync_copy(g1_hbm.at[pl.ds(base, tw)], g1v)
        cg0 = pltpu.async_copy(g0v, gs_hbm.at[p0v], sg0)
        cg1 = pltpu.async_copy(g1v, gs_hbm.at[p1v], sg1)
        rows = (rows0, rows1)
        sas = (sa0, sa1)
        sbs = (sb0, sb1)
        cps = [None, None]
        for j in range(nch):
            p = j % 2
            if cps[p] is not None:
                cps[p][0].wait()
                cps[p][1].wait()
            rv = rows[p]
            pltpu.sync_copy(xn_hbm.at[pl.ds(base + j * ch, ch)], rv)
            c0 = pltpu.async_copy(rv, xs_hbm.at[p0v.at[pl.ds(j * ch, ch)]],
                                  sas[p])
            c1 = pltpu.async_copy(rv, xs_hbm.at[p1v.at[pl.ds(j * ch, ch)]],
                                  sbs[p])
            cps[p] = (c0, c1)
        for cp in cps:
            if cp is not None:
                cp[0].wait()
                cp[1].wait()
        cg0.wait()
        cg1.wait()

    return k


def _gemm_body(be_ref, xblk_ref, act_ref,
               xs_ref, w1_ref, w3_ref, w2_ref, gs_ref, out_ref):
    b = pl.program_id(0)

    @pl.when(act_ref[b] == 1)
    def _():
        x = xs_ref[...].astype(jnp.bfloat16)
        h1 = jnp.dot(x, w1_ref[0], preferred_element_type=jnp.float32)
        h3 = jnp.dot(x, w3_ref[0], preferred_element_type=jnp.float32)
        h = ((h1 * jax.nn.sigmoid(h1)) * h3).astype(jnp.bfloat16)
        o = jnp.dot(h, w2_ref[0], preferred_element_type=jnp.float32)
        out_ref[...] = o * gs_ref[...]


def _make_combine(p_total, t_total, ch):
    tok_w = t_total // NW
    nch = tok_w // ch
    mesh = plsc.VectorSubcoreMesh(core_axis_name="c", subcore_axis_name="s")

    @functools.partial(
        pl.kernel, mesh=mesh,
        out_type=jax.ShapeDtypeStruct((t_total, D), jnp.float32),
        scratch_types=[pltpu.VMEM((TOPK * ch,), jnp.int32),
                       pltpu.VMEM((TOPK * ch, D), jnp.float32),
                       pltpu.VMEM((ch, D), jnp.float32),
                       pltpu.SemaphoreType.DMA],
    )
    def k(pos_hbm, eo_hbm, out_hbm, idx_v, rows_v, out_v, sem):
        wid = lax.axis_index("s") * NC + lax.axis_index("c")
        for j in range(nch):
            pltpu.sync_copy(
                pos_hbm.at[pl.ds(TOPK * (wid * tok_w + j * ch), TOPK * ch)], idx_v)
            pltpu.async_copy(eo_hbm.at[idx_v], rows_v, sem).wait()
            for i in range(ch):
                def add_body(g, c, i=i):
                    s = pl.ds(g * L, L)
                    out_v[i, s] = rows_v[TOPK * i, s] + rows_v[TOPK * i + 1, s]
                    return c
                lax.fori_loop(0, D // L, add_body, 0)
            pltpu.sync_copy(out_v, out_hbm.at[pl.ds(wid * tok_w + j * ch, ch)])

    return k


def kernel(inputs, Wg, W1, W3, W2):
    bs, sl, d = inputs.shape
    t = bs * sl
    a_total = t * TOPK
    nblk = a_total // BT + E
    p_total = nblk * BT
    nt = t // TB1

    x = inputs.reshape(t, d)

    gate_fn = functools.partial(_gate_body, nt=nt, t_total=t)
    xn, top2, gates, rank, cnt, load = pl.pallas_call(
        gate_fn,
        grid=(nt,),
        in_specs=[
            pl.BlockSpec((TB1, d), lambda tb: (tb, 0)),
            pl.BlockSpec((d, E), lambda tb: (0, 0)),
        ],
        out_specs=[
            pl.BlockSpec((TB1, d), lambda tb: (tb, 0)),
            pl.BlockSpec((TB1, TOPK), lambda tb: (tb, 0)),
            pl.BlockSpec((TB1, TOPK), lambda tb: (tb, 0)),
            pl.BlockSpec((TB1, TOPK), lambda tb: (tb, 0)),
            pl.BlockSpec((1, E), lambda tb: (0, 0)),
            pl.BlockSpec((1, 1), lambda tb: (0, 0)),
        ],
        out_shape=[
            jax.ShapeDtypeStruct((t, d), jnp.float32),
            jax.ShapeDtypeStruct((t, TOPK), jnp.int32),
            jax.ShapeDtypeStruct((t, TOPK), jnp.float32),
            jax.ShapeDtypeStruct((t, TOPK), jnp.int32),
            jax.ShapeDtypeStruct((1, E), jnp.float32),
            jax.ShapeDtypeStruct((1, 1), jnp.float32),
        ],
        scratch_shapes=[pltpu.VMEM((1, E), jnp.float32),
                        pltpu.VMEM((1, E), jnp.float32)],
    )(x, Wg)

    fin_fn = functools.partial(_finalize_body, nblk=nblk, t_total=t)
    pos, be, xblk, act = pl.pallas_call(
        fin_fn,
        grid=(1,),
        in_specs=[
            pl.BlockSpec((t, TOPK), lambda i: (0, 0)),
            pl.BlockSpec((t, TOPK), lambda i: (0, 0)),
            pl.BlockSpec((1, E), lambda i: (0, 0)),
        ],
        out_specs=[
            pl.BlockSpec((t, TOPK), lambda i: (0, 0)),
            pl.BlockSpec((1, nblk), lambda i: (0, 0)),
            pl.BlockSpec((1, nblk), lambda i: (0, 0)),
            pl.BlockSpec((1, nblk), lambda i: (0, 0)),
        ],
        out_shape=[
            jax.ShapeDtypeStruct((t, TOPK), jnp.int32),
            jax.ShapeDtypeStruct((1, nblk), jnp.int32),
            jax.ShapeDtypeStruct((1, nblk), jnp.int32),
            jax.ShapeDtypeStruct((1, nblk), jnp.int32),
        ],
    )(top2, rank, cnt)

    pos_flat = pos.reshape(a_total)
    gates_flat = gates.reshape(a_total)

    xs, gscale = _make_dispatch(t, p_total, 32)(
        xn, pos[:, 0], pos[:, 1], gates[:, 0], gates[:, 1])

    grid_spec = pltpu.PrefetchScalarGridSpec(
        num_scalar_prefetch=3,
        grid=(nblk,),
        in_specs=[
            pl.BlockSpec((BT, d), lambda b, be_r, xb_r, ac_r: (xb_r[b], 0)),
            pl.BlockSpec((1, d, F), lambda b, be_r, xb_r, ac_r: (be_r[b], 0, 0)),
            pl.BlockSpec((1, d, F), lambda b, be_r, xb_r, ac_r: (be_r[b], 0, 0)),
            pl.BlockSpec((1, F, d), lambda b, be_r, xb_r, ac_r: (be_r[b], 0, 0)),
            pl.BlockSpec((BT, 1), lambda b, be_r, xb_r, ac_r: (xb_r[b], 0)),
        ],
        out_specs=pl.BlockSpec((BT, d), lambda b, be_r, xb_r, ac_r: (xb_r[b], 0)),
    )
    eo = pl.pallas_call(
        _gemm_body,
        grid_spec=grid_spec,
        out_shape=jax.ShapeDtypeStruct((p_total, d), jnp.float32),
    )(be.reshape(nblk), xblk.reshape(nblk), act.reshape(nblk),
      xs, W1.astype(jnp.bfloat16), W3.astype(jnp.bfloat16),
      W2.astype(jnp.bfloat16), gscale.reshape(p_total, 1))

    if "c" in _SC:
        out = _make_combine(p_total, t, 32)(pos_flat, eo)
    else:
        out = eo[pos[:, 0]] + eo[pos[:, 1]]

    return out.reshape(bs, sl, d), load.reshape(())[()]


# speedup vs baseline: 1.1783x; 1.1783x over previous
import functools

import jax
import jax.numpy as jnp
from jax import lax
from jax.experimental import pallas as pl
from jax.experimental.pallas import tpu as pltpu
from jax.experimental.pallas import tpu_sc as plsc

E = 8
TOPK = 2
D = 1024
F = 1024
BT = 256
TB1 = 512
NC, NS, L = 2, 16, 16
NW = NC * NS
_SC = {"t", "x", "c"}


def _gate_body(x_ref, wg_ref, xn_ref, top2_ref, gates_ref, rank_ref,
               cnt_ref, load_ref, fsum_ref, psum_ref, *, nt, t_total):
    tb = pl.program_id(0)

    @pl.when(tb == 0)
    def _init():
        fsum_ref[...] = jnp.zeros_like(fsum_ref)
        psum_ref[...] = jnp.zeros_like(psum_ref)

    x = x_ref[...]
    logits = jnp.dot(x, wg_ref[...], preferred_element_type=jnp.float32)
    m = jnp.max(logits, axis=1, keepdims=True)
    ex = jnp.exp(logits - m)
    probs = ex / jnp.sum(ex, axis=1, keepdims=True)
    iota8 = lax.broadcasted_iota(jnp.int32, probs.shape, 1)
    v1 = jnp.max(probs, axis=1, keepdims=True)
    i1 = jnp.min(jnp.where(probs >= v1, iota8, E), axis=1, keepdims=True)
    p2 = jnp.where(iota8 == i1, -1.0, probs)
    v2 = jnp.max(p2, axis=1, keepdims=True)
    i2 = jnp.min(jnp.where(p2 >= v2, iota8, E), axis=1, keepdims=True)

    e_lo = jnp.minimum(i1, i2)
    e_hi = jnp.maximum(i1, i2)
    g_lo = jnp.where(i1 < i2, v1, v2)
    g_hi = jnp.where(i1 < i2, v2, v1)

    oh_lo = (iota8 == e_lo).astype(jnp.float32)
    oh_hi = (iota8 == e_hi).astype(jnp.float32)
    comb = oh_lo + oh_hi
    ri = lax.broadcasted_iota(jnp.int32, (TB1, TB1), 0)
    ci = lax.broadcasted_iota(jnp.int32, (TB1, TB1), 1)
    ltri = (ci < ri).astype(jnp.float32)
    excl = jnp.dot(ltri, comb, preferred_element_type=jnp.float32)
    base = fsum_ref[...] + excl
    rank_lo = jnp.sum(oh_lo * base, axis=1, keepdims=True)
    rank_hi = jnp.sum(oh_hi * base, axis=1, keepdims=True)

    fsum_ref[...] += jnp.sum(comb, axis=0, keepdims=True)
    psum_ref[...] += jnp.sum(probs, axis=0, keepdims=True)

    xn_ref[...] = x * lax.rsqrt(jnp.mean(x * x, axis=1, keepdims=True) + 1e-8)
    top2_ref[...] = jnp.concatenate([e_lo, e_hi], axis=1)
    gates_ref[...] = jnp.concatenate([g_lo, g_hi], axis=1)
    rank_ref[...] = jnp.concatenate([rank_lo, rank_hi], axis=1).astype(jnp.int32)

    @pl.when(tb == nt - 1)
    def _fin():
        cnt_ref[...] = fsum_ref[...]
        load_ref[...] = (E / (t_total * t_total)) * jnp.sum(
            fsum_ref[...] * psum_ref[...], axis=1, keepdims=True)


def _finalize_body(top2_ref, rank_ref, cnt_ref,
                   pos_ref, be_ref, xblk_ref, act_ref, *, nblk, t_total):
    cnt = cnt_ref[...]
    nb = jnp.floor((cnt + (BT - 1)) * (1.0 / BT))
    bstarts = []
    acc = jnp.zeros((1, 1), jnp.float32)
    for e in range(E):
        bstarts.append(acc)
        acc = acc + nb[:, e:e + 1]
    total_b = acc

    t2 = top2_ref[...]
    rank = rank_ref[...]
    poss = []
    for ci in range(TOPK):
        ec = t2[:, ci:ci + 1]
        sel = jnp.zeros((t_total, 1), jnp.float32)
        for e in range(E):
            sel += jnp.where(ec == e, bstarts[e] * BT, 0.0)
        poss.append(rank[:, ci:ci + 1] + sel.astype(jnp.int32))
    pos_ref[...] = jnp.concatenate(poss, axis=1)

    barr = lax.broadcasted_iota(jnp.int32, (1, nblk), 1).astype(jnp.float32)
    act_ref[...] = (barr < total_b).astype(jnp.int32)
    bcl = jnp.minimum(barr, total_b - 1.0)
    xblk_ref[...] = bcl.astype(jnp.int32)
    be = jnp.zeros((1, nblk), jnp.float32)
    for e in range(E):
        be += (bstarts[e] <= bcl).astype(jnp.float32)
    be_ref[...] = (be - 1.0).astype(jnp.int32)


def _make_dispatch(t_total, p_total, ch):
    tw = t_total // NW
    nch = tw // ch
    mesh = plsc.VectorSubcoreMesh(core_axis_name="c", subcore_axis_name="s")

    @functools.partial(
        pl.kernel, mesh=mesh,
        out_type=[jax.ShapeDtypeStruct((p_total, D), jnp.float32),
                  jax.ShapeDtypeStruct((p_total,), jnp.float32)],
        scratch_types=[pltpu.VMEM((tw,), jnp.int32),
                       pltpu.VMEM((tw,), jnp.int32),
                       pltpu.VMEM((tw,), jnp.float32),
                       pltpu.VMEM((tw,), jnp.float32),
                       pltpu.VMEM((ch, D), jnp.float32),
                       pltpu.VMEM((ch, D), jnp.float32),
                       pltpu.SemaphoreType.DMA,
                       pltpu.SemaphoreType.DMA,
                       pltpu.SemaphoreType.DMA,
                       pltpu.SemaphoreType.DMA,
                       pltpu.SemaphoreType.DMA,
                       pltpu.SemaphoreType.DMA],
    )
    def k(xn_hbm, pos0_hbm, pos1_hbm, g0_hbm, g1_hbm, xs_hbm, gs_hbm,
          p0v, p1v, g0v, g1v, rows0, rows1, sg0, sg1, sa0, sa1, sb0, sb1):
        wid = lax.axis_index("s") * NC + lax.axis_index("c")
        base = wid * tw
        pltpu.sync_copy(pos0_hbm.at[pl.ds(base, tw)], p0v)
        pltpu.sync_copy(pos1_hbm.at[pl.ds(base, tw)], p1v)
        pltpu.sync_copy(g0_hbm.at[pl.ds(base, tw)], g0v)
        pltpu.sync_copy(g1_hbm.at[pl.ds(base, tw)], g1v)
        cg0 = pltpu.async_copy(g0v, gs_hbm.at[p0v], sg0)
        cg1 = pltpu.async_copy(g1v, gs_hbm.at[p1v], sg1)
        rows = (rows0, rows1)
        sas = (sa0, sa1)
        sbs = (sb0, sb1)
        cps = [None, None]
        for j in range(nch):
            p = j % 2
            if cps[p] is not None:
                cps[p][0].wait()
                cps[p][1].wait()
            rv = rows[p]
            pltpu.sync_copy(xn_hbm.at[pl.ds(base + j * ch, ch)], rv)
            c0 = pltpu.async_copy(rv, xs_hbm.at[p0v.at[pl.ds(j * ch, ch)]],
                                  sas[p])
            c1 = pltpu.async_copy(rv, xs_hbm.at[p1v.at[pl.ds(j * ch, ch)]],
                                  sbs[p])
            cps[p] = (c0, c1)
        for cp in cps:
            if cp is not None:
                cp[0].wait()
                cp[1].wait()
        cg0.wait()
        cg1.wait()

    return k


def _gemm_body(be_ref, xblk_ref, act_ref,
               xs_ref, w1_ref, w3_ref, w2_ref, gs_ref, out_ref):
    b = pl.program_id(0)

    @pl.when(act_ref[b] == 1)
    def _():
        x = xs_ref[...]
        h1 = jnp.dot(x, w1_ref[0], preferred_element_type=jnp.float32)
        h3 = jnp.dot(x, w3_ref[0], preferred_element_type=jnp.float32)
        h = (h1 * jax.nn.sigmoid(h1)) * h3
        o = jnp.dot(h, w2_ref[0], preferred_element_type=jnp.float32)
        out_ref[...] = o * gs_ref[...]


def _make_combine(p_total, t_total, ch):
    tok_w = t_total // NW
    nch = tok_w // ch
    mesh = plsc.VectorSubcoreMesh(core_axis_name="c", subcore_axis_name="s")

    @functools.partial(
        pl.kernel, mesh=mesh,
        out_type=jax.ShapeDtypeStruct((t_total, D), jnp.float32),
        scratch_types=[pltpu.VMEM((TOPK * ch,), jnp.int32),
                       pltpu.VMEM((TOPK * ch,), jnp.int32),
                       pltpu.VMEM((TOPK * ch, D), jnp.float32),
                       pltpu.VMEM((TOPK * ch, D), jnp.float32),
                       pltpu.VMEM((ch, D), jnp.float32),
                       pltpu.VMEM((ch, D), jnp.float32),
                       pltpu.SemaphoreType.DMA,
                       pltpu.SemaphoreType.DMA,
                       pltpu.SemaphoreType.DMA,
                       pltpu.SemaphoreType.DMA],
    )
    def k(pos_hbm, eo_hbm, out_hbm, idx0, idx1, rows0, rows1, ob0, ob1,
          gs0, gs1, os0, os1):
        wid = lax.axis_index("s") * NC + lax.axis_index("c")
        idxs = (idx0, idx1)
        rows = (rows0, rows1)
        obufs = (ob0, ob1)
        gsems = (gs0, gs1)
        osems = (os0, os1)

        def issue(j):
            p = j % 2
            pltpu.sync_copy(
                pos_hbm.at[pl.ds(TOPK * (wid * tok_w + j * ch), TOPK * ch)],
                idxs[p])
            return pltpu.async_copy(eo_hbm.at[idxs[p]], rows[p], gsems[p])

        gcps = [None, None]
        ocps = [None, None]
        gcps[0] = issue(0)
        for j in range(nch):
            p = j % 2
            if j + 1 < nch:
                gcps[1 - p] = issue(j + 1)
            gcps[p].wait()
            if ocps[p] is not None:
                ocps[p].wait()
            rows_v = rows[p]
            out_v = obufs[p]
            for i in range(ch):
                def add_body(g, c, i=i):
                    sl = pl.ds(g * L, L)
                    out_v[i, sl] = rows_v[TOPK * i, sl] + rows_v[TOPK * i + 1, sl]
                    return c
                lax.fori_loop(0, D // L, add_body, 0)
            ocps[p] = pltpu.async_copy(
                out_v, out_hbm.at[pl.ds(wid * tok_w + j * ch, ch)], osems[p])
        for cp in ocps:
            if cp is not None:
                cp.wait()

    return k


def kernel(inputs, Wg, W1, W3, W2):
    bs, sl, d = inputs.shape
    t = bs * sl
    a_total = t * TOPK
    nblk = a_total // BT + E
    p_total = nblk * BT
    nt = t // TB1

    x = inputs.reshape(t, d)

    gate_fn = functools.partial(_gate_body, nt=nt, t_total=t)
    xn, top2, gates, rank, cnt, load = pl.pallas_call(
        gate_fn,
        grid=(nt,),
        in_specs=[
            pl.BlockSpec((TB1, d), lambda tb: (tb, 0)),
            pl.BlockSpec((d, E), lambda tb: (0, 0)),
        ],
        out_specs=[
            pl.BlockSpec((TB1, d), lambda tb: (tb, 0)),
            pl.BlockSpec((TB1, TOPK), lambda tb: (tb, 0)),
            pl.BlockSpec((TB1, TOPK), lambda tb: (tb, 0)),
            pl.BlockSpec((TB1, TOPK), lambda tb: (tb, 0)),
            pl.BlockSpec((1, E), lambda tb: (0, 0)),
            pl.BlockSpec((1, 1), lambda tb: (0, 0)),
        ],
        out_shape=[
            jax.ShapeDtypeStruct((t, d), jnp.float32),
            jax.ShapeDtypeStruct((t, TOPK), jnp.int32),
            jax.ShapeDtypeStruct((t, TOPK), jnp.float32),
            jax.ShapeDtypeStruct((t, TOPK), jnp.int32),
            jax.ShapeDtypeStruct((1, E), jnp.float32),
            jax.ShapeDtypeStruct((1, 1), jnp.float32),
        ],
        scratch_shapes=[pltpu.VMEM((1, E), jnp.float32),
                        pltpu.VMEM((1, E), jnp.float32)],
    )(x, Wg)

    fin_fn = functools.partial(_finalize_body, nblk=nblk, t_total=t)
    pos, be, xblk, act = pl.pallas_call(
        fin_fn,
        grid=(1,),
        in_specs=[
            pl.BlockSpec((t, TOPK), lambda i: (0, 0)),
            pl.BlockSpec((t, TOPK), lambda i: (0, 0)),
            pl.BlockSpec((1, E), lambda i: (0, 0)),
        ],
        out_specs=[
            pl.BlockSpec((t, TOPK), lambda i: (0, 0)),
            pl.BlockSpec((1, nblk), lambda i: (0, 0)),
            pl.BlockSpec((1, nblk), lambda i: (0, 0)),
            pl.BlockSpec((1, nblk), lambda i: (0, 0)),
        ],
        out_shape=[
            jax.ShapeDtypeStruct((t, TOPK), jnp.int32),
            jax.ShapeDtypeStruct((1, nblk), jnp.int32),
            jax.ShapeDtypeStruct((1, nblk), jnp.int32),
            jax.ShapeDtypeStruct((1, nblk), jnp.int32),
        ],
    )(top2, rank, cnt)

    pos_flat = pos.reshape(a_total)
    gates_flat = gates.reshape(a_total)

    xs, gscale = _make_dispatch(t, p_total, 32)(
        xn, pos[:, 0], pos[:, 1], gates[:, 0], gates[:, 1])

    grid_spec = pltpu.PrefetchScalarGridSpec(
        num_scalar_prefetch=3,
        grid=(nblk,),
        in_specs=[
            pl.BlockSpec((BT, d), lambda b, be_r, xb_r, ac_r: (xb_r[b], 0)),
            pl.BlockSpec((1, d, F), lambda b, be_r, xb_r, ac_r: (be_r[b], 0, 0)),
            pl.BlockSpec((1, d, F), lambda b, be_r, xb_r, ac_r: (be_r[b], 0, 0)),
            pl.BlockSpec((1, F, d), lambda b, be_r, xb_r, ac_r: (be_r[b], 0, 0)),
            pl.BlockSpec((BT, 1), lambda b, be_r, xb_r, ac_r: (xb_r[b], 0)),
        ],
        out_specs=pl.BlockSpec((BT, d), lambda b, be_r, xb_r, ac_r: (xb_r[b], 0)),
    )
    eo = pl.pallas_call(
        _gemm_body,
        grid_spec=grid_spec,
        out_shape=jax.ShapeDtypeStruct((p_total, d), jnp.float32),
    )(be.reshape(nblk), xblk.reshape(nblk), act.reshape(nblk),
      xs, W1, W3, W2, gscale.reshape(p_total, 1))

    if "c" in _SC:
        out = _make_combine(p_total, t, 16)(pos_flat, eo)
    else:
        out = eo[pos[:, 0]] + eo[pos[:, 1]]

    return out.reshape(bs, sl, d), load.reshape(())[()]
